# baseline (speedup 1.0000x reference)
import jax
import jax.numpy as jnp
from jax.experimental import pallas as pl
from jax.experimental.pallas import tpu as pltpu

_N = 50000
_D = 64
_NIDX = 10000
_ALPHA = 0.1
_NITER = 10
_CHUNK = 8192
_NP = _N // 2


def _ppr_kernel(src_ref, dst_ref, vals_ref, idx_ref, local_ref, out_ref,
                preds_ref, acc_ref):
    it = pl.program_id(0)
    c = pl.program_id(1)
    nchunk = pl.num_programs(1)

    lanes = jax.lax.broadcasted_iota(jnp.int32, (1, 2 * _D), 1)
    half = lanes // _D

    @pl.when((it == 0) & (c == 0))
    def _init():
        preds_ref[...] = local_ref[...]

    @pl.when(c == 0)
    def _start_iter():
        acc_ref[...] = _ALPHA * local_ref[...]

    def edge_body(e, carry):
        s = src_ref[e]
        d = dst_ref[e]
        v = vals_ref[e]
        s2 = s // 2
        d2 = d // 2
        ps = s & 1
        pd = d & 1
        pair = preds_ref[pl.ds(s2, 1), :]
        y = jnp.where(half == ps, pair, 0.0)
        both = y + pltpu.roll(y, _D, 1)
        contrib = jnp.where(half == pd, both, 0.0) * v
        acc_ref[pl.ds(d2, 1), :] = acc_ref[pl.ds(d2, 1), :] + contrib
        return carry

    jax.lax.fori_loop(0, _CHUNK, edge_body, 0)

    @pl.when(c == nchunk - 1)
    def _finish_iter():
        preds_ref[...] = acc_ref[...]

    @pl.when((it == _NITER - 1) & (c == nchunk - 1))
    def _gather():
        def g(i, carry):
            t = idx_ref[i]
            t2 = t // 2
            pt = t & 1
            pair = acc_ref[pl.ds(t2, 1), :]
            y = jnp.where(half == pt, pair, 0.0)
            both = y + pltpu.roll(y, _D, 1)
            out_ref[pl.ds(i, 1), :] = both[:, :_D]
            return carry

        jax.lax.fori_loop(0, _NIDX, g, 0)


def kernel(local_preds, A_vals, idx, edge_index):
    src = edge_index[0].astype(jnp.int32)
    dst = edge_index[1].astype(jnp.int32)
    idx = idx.astype(jnp.int32)
    ne = A_vals.shape[0]
    nchunk = -(-ne // _CHUNK)
    pad = nchunk * _CHUNK - ne
    if pad:
        src = jnp.pad(src, (0, pad))
        dst = jnp.pad(dst, (0, pad))
        A_vals = jnp.pad(A_vals, (0, pad))
    local_packed = local_preds.reshape(_NP, 2 * _D)
    grid = (_NITER, nchunk)
    out = pl.pallas_call(
        _ppr_kernel,
        grid=grid,
        in_specs=[
            pl.BlockSpec((_CHUNK,), lambda it, c: (c,), memory_space=pltpu.SMEM),
            pl.BlockSpec((_CHUNK,), lambda it, c: (c,), memory_space=pltpu.SMEM),
            pl.BlockSpec((_CHUNK,), lambda it, c: (c,), memory_space=pltpu.SMEM),
            pl.BlockSpec((_NIDX,), lambda it, c: (0,), memory_space=pltpu.SMEM),
            pl.BlockSpec((_NP, 2 * _D), lambda it, c: (0, 0)),
        ],
        out_specs=pl.BlockSpec((_NIDX, _D), lambda it, c: (0, 0)),
        out_shape=jax.ShapeDtypeStruct((_NIDX, _D), jnp.float32),
        scratch_shapes=[
            pltpu.VMEM((_NP, 2 * _D), jnp.float32),
            pltpu.VMEM((_NP, 2 * _D), jnp.float32),
        ],
    )(src, dst, A_vals, idx, local_packed)
    return out
